# Initial kernel scaffold; baseline (speedup 1.0000x reference)
import jax
import jax.numpy as jnp
from jax.experimental import pallas as pl


def kernel(x, edge_index, params):
    raise NotImplementedError("write your pallas kernel here")


# speedup vs baseline: 8.7382x; 8.7382x over previous
import functools

import jax
import jax.numpy as jnp
from jax import lax
from jax.experimental import pallas as pl
from jax.experimental.pallas import tpu as pltpu
from jax.experimental.pallas import tpu_sc as plsc

N = 50000
E = 800000
IN_DIM = 4
H = 64
L = 3
EPS = 1e-5

NC = 2
NS = 16
HALF = N // NC
ACC_ROWS = 25088
HIST_ROWS = 50080
CHUNK = 128

ROW_BLK = 1000
NBLK = N // ROW_BLK

def _mesh():
    return plsc.VectorSubcoreMesh(
        core_axis_name="c", subcore_axis_name="s",
        num_cores=NC, num_subcores=NS)


def _fill_rows(ref, nrows, value, width=16):
    def body(j, _):
        ref[j, :] = jnp.full((width,), value, ref.dtype)
        return 0
    lax.fori_loop(0, nrows, body, 0)


def _deg_body(dst_hbm, out_hbm, ones_v, idx_v, zeros_v, hist_sh):
    c = lax.axis_index("c")
    s = lax.axis_index("s")

    _fill_rows(ones_v, CHUNK, 1.0)
    _fill_rows(zeros_v, CHUNK, 0.0)

    nchunks = HIST_ROWS // CHUNK
    def zbody(k, _):
        cidx = s + k * NS
        @pl.when(cidx < 391)
        def _():
            pltpu.sync_copy(zeros_v, hist_sh.at[pl.ds(cidx * CHUNK, CHUNK)])
        return 0
    lax.fori_loop(0, 25, zbody, 0)
    @pl.when(s == 0)
    def _():
        pltpu.sync_copy(zeros_v.at[pl.ds(0, 32)], hist_sh.at[pl.ds(391 * CHUNK, 32)])
    plsc.subcore_barrier()

    epw = E // (NC * NS)
    wid = s * NC + c
    base = wid * epw
    nfull = epw // CHUNK
    rem = epw - nfull * CHUNK

    def ebody(k, _):
        off = base + k * CHUNK
        pltpu.sync_copy(dst_hbm.at[pl.ds(off, CHUNK)], idx_v)
        pltpu.sync_copy(ones_v, hist_sh.at[idx_v], add=True)
        return 0
    lax.fori_loop(0, nfull, ebody, 0)

    for j in range(0, CHUNK, 16):
        idx_v[pl.ds(j, 16)] = jnp.full((16,), N, jnp.int32)
    pltpu.sync_copy(dst_hbm.at[pl.ds(base + nfull * CHUNK, rem)],
                    idx_v.at[pl.ds(0, rem)])
    pltpu.sync_copy(ones_v, hist_sh.at[idx_v], add=True)

    plsc.subcore_barrier()

    def obody(k, _):
        cidx = s + k * NS
        @pl.when(cidx < 391)
        def _():
            pltpu.sync_copy(hist_sh.at[pl.ds(cidx * CHUNK, CHUNK)],
                            out_hbm.at[pl.ds(c * HIST_ROWS + cidx * CHUNK, CHUNK)])
        return 0
    lax.fori_loop(0, 25, obody, 0)
    @pl.when(s == 0)
    def _():
        pltpu.sync_copy(hist_sh.at[pl.ds(391 * CHUNK, 32)],
                        out_hbm.at[pl.ds(c * HIST_ROWS + 391 * CHUNK, 32)])


def _segsum_body(g_hbm, src_hbm, dst_hbm, out_hbm,
                 src_v, dst_v, rows_v, zeros_v, acc_sh, sem):
    c = lax.axis_index("c")
    s = lax.axis_index("s")

    def zfill(j, _):
        for jj in range(0, H, 16):
            zeros_v[j, pl.ds(jj, 16)] = jnp.zeros((16,), jnp.float32)
        return 0
    lax.fori_loop(0, CHUNK, zfill, 0)

    def zbody(k, _):
        cidx = s + k * NS
        @pl.when(cidx < 196)
        def _():
            pltpu.sync_copy(zeros_v, acc_sh.at[pl.ds(cidx * CHUNK, CHUNK)])
        return 0
    lax.fori_loop(0, 13, zbody, 0)
    plsc.subcore_barrier()

    ept = E // NS
    base = s * ept
    nfull = ept // CHUNK
    rem = ept - nfull * CHUNK
    lo = c * HALF

    def localize():
        for j in range(0, CHUNK, 16):
            d = dst_v[pl.ds(j, 16)]
            dl = d - lo
            ok = (dl >= 0) & (dl < HALF)
            dst_v[pl.ds(j, 16)] = jnp.where(ok, dl, HALF)

    def process(off, n):
        if n < CHUNK:
            for j in range(0, CHUNK, 16):
                src_v[pl.ds(j, 16)] = jnp.zeros((16,), jnp.int32)
                dst_v[pl.ds(j, 16)] = jnp.full((16,), -1, jnp.int32)
        pltpu.sync_copy(src_hbm.at[pl.ds(off, n)], src_v.at[pl.ds(0, n)])
        pltpu.sync_copy(dst_hbm.at[pl.ds(off, n)], dst_v.at[pl.ds(0, n)])
        localize()
        pltpu.async_copy(g_hbm.at[src_v], rows_v, sem).wait()
        pltpu.sync_copy(rows_v, acc_sh.at[dst_v], add=True)

    def ebody(k, _):
        process(base + k * CHUNK, CHUNK)
        return 0
    lax.fori_loop(0, nfull, ebody, 0)
    process(base + nfull * CHUNK, rem)

    plsc.subcore_barrier()

    def obody(k, _):
        cidx = s + k * NS
        @pl.when(cidx < 195)
        def _():
            r = cidx * CHUNK
            pltpu.sync_copy(acc_sh.at[pl.ds(r, CHUNK)],
                            out_hbm.at[pl.ds(lo + r, CHUNK)])
        return 0
    lax.fori_loop(0, 13, obody, 0)
    @pl.when(s == 0)
    def _():
        r = 195 * CHUNK
        pltpu.sync_copy(acc_sh.at[pl.ds(r, 40)], out_hbm.at[pl.ds(lo + r, 40)])


_SC_CACHE = {}


def _deg_kernel(dst):
    if "deg" not in _SC_CACHE:
        _SC_CACHE["deg"] = pl.kernel(
            _deg_body,
            out_type=jax.ShapeDtypeStruct((NC * HIST_ROWS, 16), jnp.float32),
            mesh=_mesh(),
            scratch_types=[
                pltpu.VMEM((CHUNK, 16), jnp.float32),
                pltpu.VMEM((CHUNK,), jnp.int32),
                pltpu.VMEM((CHUNK, 16), jnp.float32),
                pltpu.VMEM_SHARED((HIST_ROWS, 16), jnp.float32),
            ],
            compiler_params=pltpu.CompilerParams(use_tc_tiling_on_sc=False),
        )
    return _SC_CACHE["deg"](dst).reshape(NC, HIST_ROWS, 16)


def _segsum_kernel(g, src, dst):
    if "seg" not in _SC_CACHE:
        _SC_CACHE["seg"] = pl.kernel(
            _segsum_body,
            out_type=jax.ShapeDtypeStruct((N, H), jnp.float32),
            mesh=_mesh(),
            scratch_types=[
                pltpu.VMEM((CHUNK,), jnp.int32),
                pltpu.VMEM((CHUNK,), jnp.int32),
                pltpu.VMEM((CHUNK, H), jnp.float32),
                pltpu.VMEM((CHUNK, H), jnp.float32),
                pltpu.VMEM_SHARED((ACC_ROWS, H), jnp.float32),
                pltpu.SemaphoreType.DMA,
            ],
            compiler_params=pltpu.CompilerParams(use_tc_tiling_on_sc=False),
        )
    return _SC_CACHE["seg"](g, src, dst)


def _elu(x):
    return jnp.where(x > 0, x, jnp.exp(jnp.minimum(x, 0.0)) - 1.0)


def _k2a(x_ref, w_ref, b_ref, degp_ref, u_ref, dis_ref, sum_ref, ss_ref):
    i = pl.program_id(0)
    u = jnp.dot(x_ref[...], w_ref[...], preferred_element_type=jnp.float32)
    u = u + b_ref[...]
    u_ref[...] = u
    deg = degp_ref[0, :, 0:1] + degp_ref[1, :, 0:1] + 1.0
    dis_ref[...] = lax.rsqrt(deg)
    @pl.when(i == 0)
    def _():
        sum_ref[...] = jnp.zeros_like(sum_ref)
        ss_ref[...] = jnp.zeros_like(ss_ref)
    sum_ref[...] += jnp.sum(u, axis=0, keepdims=True)
    ss_ref[...] += jnp.sum(u * u, axis=0, keepdims=True)


def _k2b(u_ref, sum_ref, ss_ref, g_ref, b_ref, w1_ref, dis_ref, out_ref):
    m = sum_ref[...] / N
    v = ss_ref[...] / N - m * m
    inv = lax.rsqrt(v + EPS)
    h0 = _elu((u_ref[...] - m) * inv * g_ref[...] + b_ref[...])
    out_ref[...] = jnp.dot(h0, w1_ref[...],
                           preferred_element_type=jnp.float32) * dis_ref[...]


def _k5a(s_ref, g_ref, dis_ref, b_ref, t_ref, sum_ref, ss_ref):
    i = pl.program_id(0)
    t = (s_ref[...] + g_ref[...]) * dis_ref[...] + b_ref[...]
    t_ref[...] = t
    @pl.when(i == 0)
    def _():
        sum_ref[...] = jnp.zeros_like(sum_ref)
        ss_ref[...] = jnp.zeros_like(ss_ref)
    sum_ref[...] += jnp.sum(t, axis=0, keepdims=True)
    ss_ref[...] += jnp.sum(t * t, axis=0, keepdims=True)


def _k5b(t_ref, sum_ref, ss_ref, g_ref, b_ref, wn_ref, dis_ref, wo_ref,
         gn_ref, p_ref):
    m = sum_ref[...] / N
    v = ss_ref[...] / N - m * m
    inv = lax.rsqrt(v + EPS)
    h = _elu((t_ref[...] - m) * inv * g_ref[...] + b_ref[...])
    gn_ref[...] = jnp.dot(h, wn_ref[...],
                          preferred_element_type=jnp.float32) * dis_ref[...]
    p_ref[...] = jnp.dot(h, wo_ref[...], preferred_element_type=jnp.float32)


def _k5b3(t_ref, sum_ref, ss_ref, g_ref, b_ref, wo_ref, p1_ref, p2_ref,
          bo_ref, out_ref):
    m = sum_ref[...] / N
    v = ss_ref[...] / N - m * m
    inv = lax.rsqrt(v + EPS)
    h = _elu((t_ref[...] - m) * inv * g_ref[...] + b_ref[...])
    p3 = jnp.dot(h, wo_ref[...], preferred_element_type=jnp.float32)
    out_ref[...] = p1_ref[...] + p2_ref[...] + p3 + bo_ref[...]


def _rows(i):
    return (i, 0)


_SPEC_MAT = pl.BlockSpec((ROW_BLK, H), _rows)
_SPEC_VEC = pl.BlockSpec((ROW_BLK, 1), _rows)
_SPEC_STAT = pl.BlockSpec((1, H), lambda i: (0, 0))
_SPEC_H = pl.BlockSpec((H,), lambda i: (0,))
_SPEC_W = pl.BlockSpec((H, H), lambda i: (0, 0))
_SPEC_WO = pl.BlockSpec((H, 1), lambda i: (0, 0))

_f32 = jnp.float32


def _call_k2a(x, w_in, b_in, degp):
    return pl.pallas_call(
        _k2a,
        grid=(NBLK,),
        in_specs=[
            pl.BlockSpec((ROW_BLK, IN_DIM), _rows),
            pl.BlockSpec((IN_DIM, H), lambda i: (0, 0)),
            _SPEC_H,
            pl.BlockSpec((NC, ROW_BLK, 16), lambda i: (0, i, 0)),
        ],
        out_specs=[_SPEC_MAT, _SPEC_VEC, _SPEC_STAT, _SPEC_STAT],
        out_shape=[
            jax.ShapeDtypeStruct((N, H), _f32),
            jax.ShapeDtypeStruct((N, 1), _f32),
            jax.ShapeDtypeStruct((1, H), _f32),
            jax.ShapeDtypeStruct((1, H), _f32),
        ],
    )(x, w_in, b_in, degp)


def _call_k2b(u, su, ss, bn_g, bn_b, w1, dis):
    return pl.pallas_call(
        _k2b,
        grid=(NBLK,),
        in_specs=[_SPEC_MAT, _SPEC_STAT, _SPEC_STAT, _SPEC_H, _SPEC_H,
                  _SPEC_W, _SPEC_VEC],
        out_specs=_SPEC_MAT,
        out_shape=jax.ShapeDtypeStruct((N, H), _f32),
    )(u, su, ss, bn_g, bn_b, w1, dis)


def _call_k5a(s_agg, g, dis, b):
    return pl.pallas_call(
        _k5a,
        grid=(NBLK,),
        in_specs=[_SPEC_MAT, _SPEC_MAT, _SPEC_VEC, _SPEC_H],
        out_specs=[_SPEC_MAT, _SPEC_STAT, _SPEC_STAT],
        out_shape=[
            jax.ShapeDtypeStruct((N, H), _f32),
            jax.ShapeDtypeStruct((1, H), _f32),
            jax.ShapeDtypeStruct((1, H), _f32),
        ],
    )(s_agg, g, dis, b)


def _call_k5b(t, su, ss, bn_g, bn_b, wn, dis, wo):
    return pl.pallas_call(
        _k5b,
        grid=(NBLK,),
        in_specs=[_SPEC_MAT, _SPEC_STAT, _SPEC_STAT, _SPEC_H, _SPEC_H,
                  _SPEC_W, _SPEC_VEC, _SPEC_WO],
        out_specs=[_SPEC_MAT, _SPEC_VEC],
        out_shape=[
            jax.ShapeDtypeStruct((N, H), _f32),
            jax.ShapeDtypeStruct((N, 1), _f32),
        ],
    )(t, su, ss, bn_g, bn_b, wn, dis, wo)


def _call_k5b3(t, su, ss, bn_g, bn_b, wo, p1, p2, bo):
    return pl.pallas_call(
        _k5b3,
        grid=(NBLK,),
        in_specs=[_SPEC_MAT, _SPEC_STAT, _SPEC_STAT, _SPEC_H, _SPEC_H,
                  _SPEC_WO, _SPEC_VEC, _SPEC_VEC,
                  pl.BlockSpec((1, 1), lambda i: (0, 0))],
        out_specs=_SPEC_VEC,
        out_shape=jax.ShapeDtypeStruct((N, 1), _f32),
    )(t, su, ss, bn_g, bn_b, wo, p1, p2, bo)


def kernel(x, edge_index, params):
    src = edge_index[0]
    dst = edge_index[1]

    degp = _deg_kernel(dst)
    u, dis, su, ss = _call_k2a(x, params["W_in"], params["b_in"], degp)
    g = _call_k2b(u, su, ss, params["bn_in_g"], params["bn_in_b"],
                  params["conv_W"][0], dis)

    ps = []
    for i in range(L):
        s_agg = _segsum_kernel(g, src, dst)
        t, su, ss = _call_k5a(s_agg, g, dis, params["conv_b"][i])
        wo = lax.slice(params["W_out"], (i * H, 0), ((i + 1) * H, 1))
        if i < L - 1:
            g, p = _call_k5b(t, su, ss, params["bn_g"][i], params["bn_b"][i],
                             params["conv_W"][i + 1], dis, wo)
            ps.append(p)
        else:
            out = _call_k5b3(t, su, ss, params["bn_g"][i], params["bn_b"][i],
                             wo, ps[0], ps[1],
                             params["b_out"].reshape(1, 1))
    return out[:, 0]


# speedup vs baseline: 12.9056x; 1.4769x over previous
import functools

import jax
import jax.numpy as jnp
from jax import lax
from jax.experimental import pallas as pl
from jax.experimental.pallas import tpu as pltpu
from jax.experimental.pallas import tpu_sc as plsc

N = 50000
E = 800000
IN_DIM = 4
H = 64
L = 3
EPS = 1e-5

NC = 2
NS = 16
HALF = N // NC
ACC_ROWS = 25088
HIST_ROWS = 50080
CHUNK = 128

ROW_BLK = 1000
NBLK = N // ROW_BLK

def _mesh():
    return plsc.VectorSubcoreMesh(
        core_axis_name="c", subcore_axis_name="s",
        num_cores=NC, num_subcores=NS)


def _fill_rows(ref, nrows, value, width=16):
    def body(j, _):
        ref[j, :] = jnp.full((width,), value, ref.dtype)
        return 0
    lax.fori_loop(0, nrows, body, 0)


def _deg_body(dst_hbm, out_hbm, ones_v, idx_v, zeros_v, hist_sh):
    c = lax.axis_index("c")
    s = lax.axis_index("s")

    _fill_rows(ones_v, CHUNK, 1.0)
    _fill_rows(zeros_v, CHUNK, 0.0)

    nchunks = HIST_ROWS // CHUNK
    def zbody(k, _):
        cidx = s + k * NS
        @pl.when(cidx < 391)
        def _():
            pltpu.sync_copy(zeros_v, hist_sh.at[pl.ds(cidx * CHUNK, CHUNK)])
        return 0
    lax.fori_loop(0, 25, zbody, 0)
    @pl.when(s == 0)
    def _():
        pltpu.sync_copy(zeros_v.at[pl.ds(0, 32)], hist_sh.at[pl.ds(391 * CHUNK, 32)])
    plsc.subcore_barrier()

    epw = E // (NC * NS)
    wid = s * NC + c
    base = wid * epw
    nfull = epw // CHUNK
    rem = epw - nfull * CHUNK

    def ebody(k, _):
        off = base + k * CHUNK
        pltpu.sync_copy(dst_hbm.at[pl.ds(off, CHUNK)], idx_v)
        pltpu.sync_copy(ones_v, hist_sh.at[idx_v], add=True)
        return 0
    lax.fori_loop(0, nfull, ebody, 0)

    for j in range(0, CHUNK, 16):
        idx_v[pl.ds(j, 16)] = jnp.full((16,), N, jnp.int32)
    pltpu.sync_copy(dst_hbm.at[pl.ds(base + nfull * CHUNK, rem)],
                    idx_v.at[pl.ds(0, rem)])
    pltpu.sync_copy(ones_v, hist_sh.at[idx_v], add=True)

    plsc.subcore_barrier()

    def obody(k, _):
        cidx = s + k * NS
        @pl.when(cidx < 391)
        def _():
            pltpu.sync_copy(hist_sh.at[pl.ds(cidx * CHUNK, CHUNK)],
                            out_hbm.at[pl.ds(c * HIST_ROWS + cidx * CHUNK, CHUNK)])
        return 0
    lax.fori_loop(0, 25, obody, 0)
    @pl.when(s == 0)
    def _():
        pltpu.sync_copy(hist_sh.at[pl.ds(391 * CHUNK, 32)],
                        out_hbm.at[pl.ds(c * HIST_ROWS + 391 * CHUNK, 32)])


GRP = 1
NGRP = 390 // GRP
NROWS2D = E // CHUNK


def _segsum_body(g_hbm, src_hbm, dst_hbm, out_hbm,
                 src_v, dst_v, rows_v, zeros_v, acc_sh,
                 sem_g, sem_s0, sem_s1):
    c = lax.axis_index("c")
    s = lax.axis_index("s")

    def zfill(j, _):
        for jj in range(0, H, 16):
            zeros_v[j, pl.ds(jj, 16)] = jnp.zeros((16,), jnp.float32)
        return 0
    lax.fori_loop(0, 32, zfill, 0)

    def zbody(k, _):
        cidx = s + k * NS
        @pl.when(cidx < 784)
        def _():
            pltpu.sync_copy(zeros_v, acc_sh.at[pl.ds(cidx * 32, 32)])
        return 0
    lax.fori_loop(0, 49, zbody, 0)
    plsc.subcore_barrier()

    lo = c * HALF
    r0 = s * (GRP * NGRP)

    def emit(kt, slot, half, do):
        rows_h = rows_v.at[half]
        sem_s = sem_s0 if half == 0 else sem_s1
        if "L" in do:
            @pl.when((kt >= 0) & (kt < NGRP))
            def _():
                pltpu.sync_copy(src_hbm.at[pl.ds(r0 + kt * GRP, GRP)],
                                src_v.at[slot])
                pltpu.sync_copy(dst_hbm.at[pl.ds(r0 + kt * GRP, GRP)],
                                dst_v.at[slot])
        if "loc" in do:
            @pl.when((kt >= 0) & (kt < NGRP))
            def _():
                for b in range(GRP):
                    for j in range(0, CHUNK, 16):
                        d = dst_v[slot, b, pl.ds(j, 16)]
                        dl = d - lo
                        ok = (dl >= 0) & (dl < HALF)
                        dst_v[slot, b, pl.ds(j, 16)] = jnp.where(ok, dl, HALF)
        if "G" in do:
            @pl.when((kt >= 0) & (kt < NGRP))
            def _():
                for b in range(GRP):
                    pltpu.async_copy(g_hbm.at[src_v.at[slot, b]],
                                     rows_h.at[pl.ds(b * CHUNK, CHUNK)], sem_g)
        if "W" in do:
            @pl.when((kt >= 0) & (kt < NGRP))
            def _():
                for b in range(GRP):
                    pltpu.make_async_copy(
                        g_hbm.at[src_v.at[slot, b]],
                        rows_h.at[pl.ds(b * CHUNK, CHUNK)], sem_g).wait()
        if "S" in do:
            @pl.when((kt >= 0) & (kt < NGRP))
            def _():
                for b in range(GRP):
                    pltpu.async_copy(rows_h.at[pl.ds(b * CHUNK, CHUNK)],
                                     acc_sh.at[dst_v.at[slot, b]], sem_s,
                                     add=True)
        if "D" in do:
            @pl.when((kt >= 0) & (kt < NGRP))
            def _():
                for b in range(GRP):
                    pltpu.make_async_copy(
                        rows_h.at[pl.ds(b * CHUNK, CHUNK)],
                        acc_sh.at[dst_v.at[slot, b]], sem_s).wait()

    emit(jnp.int32(0), 0, 0, ("L", "loc", "G"))
    emit(jnp.int32(1), 1, 1, ("L",))

    def step(p, _):
        for jj in range(4):
            k = 4 * p + jj
            emit(k, jj, jj % 2, ("W", "S"))
            emit(k - 1, (jj - 1) % 4, (jj - 1) % 2, ("D",))
            emit(k + 1, (jj + 1) % 4, (jj + 1) % 2, ("loc", "G"))
            emit(k + 2, (jj + 2) % 4, (jj + 2) % 2, ("L",))
        return 0
    lax.fori_loop(0, (NGRP + 3) // 4, step, 0)

    nextra = NROWS2D - NS * GRP * NGRP
    @pl.when(s < nextra)
    def _():
        pltpu.sync_copy(src_hbm.at[pl.ds(NS * GRP * NGRP + s, 1)],
                        src_v.at[0, pl.ds(0, 1)])
        pltpu.sync_copy(dst_hbm.at[pl.ds(NS * GRP * NGRP + s, 1)],
                        dst_v.at[0, pl.ds(0, 1)])
        for j in range(0, CHUNK, 16):
            d = dst_v[0, 0, pl.ds(j, 16)]
            dl = d - lo
            ok = (dl >= 0) & (dl < HALF)
            dst_v[0, 0, pl.ds(j, 16)] = jnp.where(ok, dl, HALF)
        pltpu.async_copy(g_hbm.at[src_v.at[0, 0]],
                         rows_v.at[0, pl.ds(0, CHUNK)], sem_g).wait()
        pltpu.sync_copy(rows_v.at[0, pl.ds(0, CHUNK)],
                        acc_sh.at[dst_v.at[0, 0]], add=True)

    plsc.subcore_barrier()

    def obody(k, _):
        cidx = s + k * NS
        @pl.when(cidx < 195)
        def _():
            r = cidx * CHUNK
            pltpu.sync_copy(acc_sh.at[pl.ds(r, CHUNK)],
                            out_hbm.at[pl.ds(lo + r, CHUNK)])
        return 0
    lax.fori_loop(0, 13, obody, 0)
    @pl.when(s == 0)
    def _():
        r = 195 * CHUNK
        pltpu.sync_copy(acc_sh.at[pl.ds(r, 40)], out_hbm.at[pl.ds(lo + r, 40)])


_SC_CACHE = {}


def _deg_kernel(dst):
    if "deg" not in _SC_CACHE:
        _SC_CACHE["deg"] = pl.kernel(
            _deg_body,
            out_type=jax.ShapeDtypeStruct((NC * HIST_ROWS, 16), jnp.float32),
            mesh=_mesh(),
            scratch_types=[
                pltpu.VMEM((CHUNK, 16), jnp.float32),
                pltpu.VMEM((CHUNK,), jnp.int32),
                pltpu.VMEM((CHUNK, 16), jnp.float32),
                pltpu.VMEM_SHARED((HIST_ROWS, 16), jnp.float32),
            ],
            compiler_params=pltpu.CompilerParams(use_tc_tiling_on_sc=False),
        )
    return _SC_CACHE["deg"](dst).reshape(NC, HIST_ROWS, 16)


def _segsum_kernel(g, src, dst):
    if "seg" not in _SC_CACHE:
        _SC_CACHE["seg"] = pl.kernel(
            _segsum_body,
            out_type=jax.ShapeDtypeStruct((N, H), jnp.float32),
            mesh=_mesh(),
            scratch_types=[
                pltpu.VMEM((4, GRP, CHUNK), jnp.int32),
                pltpu.VMEM((4, GRP, CHUNK), jnp.int32),
                pltpu.VMEM((2, GRP * CHUNK, H), jnp.float32),
                pltpu.VMEM((32, H), jnp.float32),
                pltpu.VMEM_SHARED((ACC_ROWS, H), jnp.float32),
                pltpu.SemaphoreType.DMA,
                pltpu.SemaphoreType.DMA,
                pltpu.SemaphoreType.DMA,
            ],
            compiler_params=pltpu.CompilerParams(use_tc_tiling_on_sc=False),
        )
    return _SC_CACHE["seg"](g, src.reshape(NROWS2D, CHUNK),
                            dst.reshape(NROWS2D, CHUNK))


def _elu(x):
    return jnp.where(x > 0, x, jnp.exp(jnp.minimum(x, 0.0)) - 1.0)


def _k2a(x_ref, w_ref, b_ref, degp_ref, u_ref, dis_ref, sum_ref, ss_ref):
    i = pl.program_id(0)
    u = jnp.dot(x_ref[...], w_ref[...], preferred_element_type=jnp.float32)
    u = u + b_ref[...]
    u_ref[...] = u
    deg = degp_ref[0, :, 0:1] + degp_ref[1, :, 0:1] + 1.0
    dis_ref[...] = lax.rsqrt(deg)
    @pl.when(i == 0)
    def _():
        sum_ref[...] = jnp.zeros_like(sum_ref)
        ss_ref[...] = jnp.zeros_like(ss_ref)
    sum_ref[...] += jnp.sum(u, axis=0, keepdims=True)
    ss_ref[...] += jnp.sum(u * u, axis=0, keepdims=True)


def _k2b(u_ref, sum_ref, ss_ref, g_ref, b_ref, w1_ref, dis_ref, out_ref):
    m = sum_ref[...] / N
    v = ss_ref[...] / N - m * m
    inv = lax.rsqrt(v + EPS)
    h0 = _elu((u_ref[...] - m) * inv * g_ref[...] + b_ref[...])
    out_ref[...] = jnp.dot(h0, w1_ref[...],
                           preferred_element_type=jnp.float32) * dis_ref[...]


def _k5a(s_ref, g_ref, dis_ref, b_ref, t_ref, sum_ref, ss_ref):
    i = pl.program_id(0)
    t = (s_ref[...] + g_ref[...]) * dis_ref[...] + b_ref[...]
    t_ref[...] = t
    @pl.when(i == 0)
    def _():
        sum_ref[...] = jnp.zeros_like(sum_ref)
        ss_ref[...] = jnp.zeros_like(ss_ref)
    sum_ref[...] += jnp.sum(t, axis=0, keepdims=True)
    ss_ref[...] += jnp.sum(t * t, axis=0, keepdims=True)


def _k5b(t_ref, sum_ref, ss_ref, g_ref, b_ref, wn_ref, dis_ref, wo_ref,
         gn_ref, p_ref):
    m = sum_ref[...] / N
    v = ss_ref[...] / N - m * m
    inv = lax.rsqrt(v + EPS)
    h = _elu((t_ref[...] - m) * inv * g_ref[...] + b_ref[...])
    gn_ref[...] = jnp.dot(h, wn_ref[...],
                          preferred_element_type=jnp.float32) * dis_ref[...]
    p_ref[...] = jnp.dot(h, wo_ref[...], preferred_element_type=jnp.float32)


def _k5b3(t_ref, sum_ref, ss_ref, g_ref, b_ref, wo_ref, p1_ref, p2_ref,
          bo_ref, out_ref):
    m = sum_ref[...] / N
    v = ss_ref[...] / N - m * m
    inv = lax.rsqrt(v + EPS)
    h = _elu((t_ref[...] - m) * inv * g_ref[...] + b_ref[...])
    p3 = jnp.dot(h, wo_ref[...], preferred_element_type=jnp.float32)
    out_ref[...] = p1_ref[...] + p2_ref[...] + p3 + bo_ref[...]


def _rows(i):
    return (i, 0)


_SPEC_MAT = pl.BlockSpec((ROW_BLK, H), _rows)
_SPEC_VEC = pl.BlockSpec((ROW_BLK, 1), _rows)
_SPEC_STAT = pl.BlockSpec((1, H), lambda i: (0, 0))
_SPEC_H = pl.BlockSpec((H,), lambda i: (0,))
_SPEC_W = pl.BlockSpec((H, H), lambda i: (0, 0))
_SPEC_WO = pl.BlockSpec((H, 1), lambda i: (0, 0))

_f32 = jnp.float32


def _call_k2a(x, w_in, b_in, degp):
    return pl.pallas_call(
        _k2a,
        grid=(NBLK,),
        in_specs=[
            pl.BlockSpec((ROW_BLK, IN_DIM), _rows),
            pl.BlockSpec((IN_DIM, H), lambda i: (0, 0)),
            _SPEC_H,
            pl.BlockSpec((NC, ROW_BLK, 16), lambda i: (0, i, 0)),
        ],
        out_specs=[_SPEC_MAT, _SPEC_VEC, _SPEC_STAT, _SPEC_STAT],
        out_shape=[
            jax.ShapeDtypeStruct((N, H), _f32),
            jax.ShapeDtypeStruct((N, 1), _f32),
            jax.ShapeDtypeStruct((1, H), _f32),
            jax.ShapeDtypeStruct((1, H), _f32),
        ],
    )(x, w_in, b_in, degp)


def _call_k2b(u, su, ss, bn_g, bn_b, w1, dis):
    return pl.pallas_call(
        _k2b,
        grid=(NBLK,),
        in_specs=[_SPEC_MAT, _SPEC_STAT, _SPEC_STAT, _SPEC_H, _SPEC_H,
                  _SPEC_W, _SPEC_VEC],
        out_specs=_SPEC_MAT,
        out_shape=jax.ShapeDtypeStruct((N, H), _f32),
    )(u, su, ss, bn_g, bn_b, w1, dis)


def _call_k5a(s_agg, g, dis, b):
    return pl.pallas_call(
        _k5a,
        grid=(NBLK,),
        in_specs=[_SPEC_MAT, _SPEC_MAT, _SPEC_VEC, _SPEC_H],
        out_specs=[_SPEC_MAT, _SPEC_STAT, _SPEC_STAT],
        out_shape=[
            jax.ShapeDtypeStruct((N, H), _f32),
            jax.ShapeDtypeStruct((1, H), _f32),
            jax.ShapeDtypeStruct((1, H), _f32),
        ],
    )(s_agg, g, dis, b)


def _call_k5b(t, su, ss, bn_g, bn_b, wn, dis, wo):
    return pl.pallas_call(
        _k5b,
        grid=(NBLK,),
        in_specs=[_SPEC_MAT, _SPEC_STAT, _SPEC_STAT, _SPEC_H, _SPEC_H,
                  _SPEC_W, _SPEC_VEC, _SPEC_WO],
        out_specs=[_SPEC_MAT, _SPEC_VEC],
        out_shape=[
            jax.ShapeDtypeStruct((N, H), _f32),
            jax.ShapeDtypeStruct((N, 1), _f32),
        ],
    )(t, su, ss, bn_g, bn_b, wn, dis, wo)


def _call_k5b3(t, su, ss, bn_g, bn_b, wo, p1, p2, bo):
    return pl.pallas_call(
        _k5b3,
        grid=(NBLK,),
        in_specs=[_SPEC_MAT, _SPEC_STAT, _SPEC_STAT, _SPEC_H, _SPEC_H,
                  _SPEC_WO, _SPEC_VEC, _SPEC_VEC,
                  pl.BlockSpec((1, 1), lambda i: (0, 0))],
        out_specs=_SPEC_VEC,
        out_shape=jax.ShapeDtypeStruct((N, 1), _f32),
    )(t, su, ss, bn_g, bn_b, wo, p1, p2, bo)


def kernel(x, edge_index, params):
    src = edge_index[0]
    dst = edge_index[1]

    degp = _deg_kernel(dst)
    u, dis, su, ss = _call_k2a(x, params["W_in"], params["b_in"], degp)
    g = _call_k2b(u, su, ss, params["bn_in_g"], params["bn_in_b"],
                  params["conv_W"][0], dis)

    ps = []
    for i in range(L):
        s_agg = _segsum_kernel(g, src, dst)
        t, su, ss = _call_k5a(s_agg, g, dis, params["conv_b"][i])
        wo = lax.slice(params["W_out"], (i * H, 0), ((i + 1) * H, 1))
        if i < L - 1:
            g, p = _call_k5b(t, su, ss, params["bn_g"][i], params["bn_b"][i],
                             params["conv_W"][i + 1], dis, wo)
            ps.append(p)
        else:
            out = _call_k5b3(t, su, ss, params["bn_g"][i], params["bn_b"][i],
                             wo, ps[0], ps[1],
                             params["b_out"].reshape(1, 1))
    return out[:, 0]
